# baseline (speedup 1.0000x reference)
import functools

import jax
import jax.numpy as jnp
from jax import lax
from jax.experimental import pallas as pl
from jax.experimental.pallas import tpu as pltpu
from jax.experimental.pallas import tpu_sc as plsc

B, C, H, W = 4, 256, 64, 64
NH, NP, NLAYER = 8, 4, 2
HD = C // NH
N = H * W
K = NP * 4
QB = 512
QC = 64
NCHUNK = N // QC
NBH = B * NH

_PREC = lax.Precision.DEFAULT


def _dot(a, b):
    return jnp.dot(a, b, preferred_element_type=jnp.float32, precision=_PREC)


def _pos_body(w1x_r, w1y_r, g_r, b_r, w2_r, o_r):
    p = lax.broadcasted_iota(jnp.int32, (N, 1), 0)
    xs = (p % W).astype(jnp.float32) * (1.0 / W)
    ys = (p // W).astype(jnp.float32) * (1.0 / H)
    x = xs * w1x_r[...] + ys * w1y_r[...]
    mu = jnp.mean(x, 0, keepdims=True)
    var = jnp.mean((x - mu) ** 2, 0, keepdims=True)
    x = (x - mu) / jnp.sqrt(var + 1e-5) * g_r[...] + b_r[...]
    x = jnp.maximum(x, 0.0)
    o_r[...] = _dot(x, w2_r[...])


def _posembed(params):
    w1 = params["pos_w1"]
    return pl.pallas_call(
        _pos_body,
        out_shape=jax.ShapeDtypeStruct((N, C), jnp.float32),
    )(w1[0:1], w1[1:2], params["pos_bn_g"].reshape(1, C),
      params["pos_bn_b"].reshape(1, C), params["pos_w2"])


def _vproj_body(val4_r, vw_r, vb_r, vT_r):
    for j in range(4):
        vT_r[:, pl.ds(j * HD, HD)] = (
            _dot(val4_r[0][:, j * C:(j + 1) * C], vw_r[0]) + vb_r[0])


def _vproj(value4, vw, vb):
    vwT = vw.reshape(C, NH, HD).transpose(1, 0, 2)
    vbT = vb.reshape(NH, 1, HD)
    return pl.pallas_call(
        _vproj_body,
        grid=(B, N // QB, NH),
        in_specs=[
            pl.BlockSpec((1, QB // 4, 4 * C), lambda b, i, h: (b, i, 0)),
            pl.BlockSpec((1, C, HD), lambda b, i, h: (h, 0, 0)),
            pl.BlockSpec((1, 1, HD), lambda b, i, h: (h, 0, 0)),
        ],
        out_specs=pl.BlockSpec(
            (QB // 4, 128),
            lambda b, i, h: (b * (NH * N // QB) + h * (N // QB) + i, 0)),
        out_shape=jax.ShapeDtypeStruct((NBH * N // 4, 128), jnp.float32),
        compiler_params=pltpu.CompilerParams(
            dimension_semantics=("parallel", "parallel", "arbitrary")),
    )(value4, vwT, vbT)


def _dotT(wT, x):
    return lax.dot_general(wT, x, (((1,), (1,)), ((), ())),
                           preferred_element_type=jnp.float32,
                           precision=_PREC)


def _stage1_body(q_r, qp_r, owx_r, owy_r, obx_r, oby_r, aww_r, awb_r,
                 idx_r, wgt_r):
    bi = pl.program_id(0)
    ib = pl.program_id(1)
    qp = q_r[0] + qp_r[...]
    offx = _dotT(owx_r[...], qp) + obx_r[...]
    offy = _dotT(owy_r[...], qp) + oby_r[...]
    lg = _dotT(aww_r[...], qp) + awb_r[...]
    m = jnp.max(lg, axis=0, keepdims=True)
    e = jnp.exp(lg - m)
    gi = lax.broadcasted_iota(jnp.int32, (32, 32), 0) // NP
    gj = lax.broadcasted_iota(jnp.int32, (32, 32), 1) // NP
    gmat = (gi == gj).astype(jnp.float32)
    aw = e / _dot(gmat, e)

    p = ib * QB + lax.broadcasted_iota(jnp.int32, (1, QB), 1)
    xsf = (p % W).astype(jnp.float32)
    ysf = (p // W).astype(jnp.float32)
    x = xsf + offx - 0.5
    y = ysf + offy - 0.5
    x0 = jnp.floor(x)
    y0 = jnp.floor(y)
    wx1 = x - x0
    wx0 = 1.0 - wx1
    wy1 = y - y0
    wy0 = 1.0 - wy1
    idx_c = []
    wgt_c = []
    for dy, wy in ((0.0, wy0), (1.0, wy1)):
        for dx, wx in ((0.0, wx0), (1.0, wx1)):
            cx = x0 + dx
            cy = y0 + dy
            valid = ((cx >= 0.0) & (cx <= W - 1) & (cy >= 0.0) & (cy <= H - 1))
            cxi = jnp.clip(cx, 0.0, W - 1).astype(jnp.int32)
            cyi = jnp.clip(cy, 0.0, H - 1).astype(jnp.int32)
            idx_c.append(cyi * W + cxi)
            wgt_c.append(jnp.where(valid, wx * wy * aw, 0.0))
    for h in range(NH):
        s = slice(h * NP, (h + 1) * NP)
        base = (bi * NH + h) * N
        idx_r[h] = jnp.concatenate([t[s] for t in idx_c], axis=0) + base
        wgt_r[h] = jnp.concatenate([t[s] for t in wgt_c], axis=0)


def _stage1(q, q_pose, owx, owy, obx, oby, aww, awb):
    nb = N // QB
    return pl.pallas_call(
        _stage1_body,
        grid=(B, nb),
        in_specs=[
            pl.BlockSpec((1, QB, C), lambda b, i: (b, i, 0)),
            pl.BlockSpec((QB, C), lambda b, i: (i, 0)),
            pl.BlockSpec((32, C), lambda b, i: (0, 0)),
            pl.BlockSpec((32, C), lambda b, i: (0, 0)),
            pl.BlockSpec((32, 1), lambda b, i: (0, 0)),
            pl.BlockSpec((32, 1), lambda b, i: (0, 0)),
            pl.BlockSpec((32, C), lambda b, i: (0, 0)),
            pl.BlockSpec((32, 1), lambda b, i: (0, 0)),
        ],
        out_specs=[
            pl.BlockSpec((NH, K, QB), lambda b, i: (b, 0, i)),
            pl.BlockSpec((NH, K, QB), lambda b, i: (b, 0, i)),
        ],
        out_shape=[
            jax.ShapeDtypeStruct((NBH, K, N), jnp.int32),
            jax.ShapeDtypeStruct((NBH, K, N), jnp.float32),
        ],
        compiler_params=pltpu.CompilerParams(
            dimension_semantics=("parallel", "parallel")),
    )(q, q_pose, owx, owy, obx, oby, aww, awb)


def _sc_body(vT_r, idxh_r, wgth_r, out_r, idx_v, wgt_v, rows_v, out_v,
             sem_iw, sem_gat, sem_out):
    wid = lax.axis_index("s") * 2 + lax.axis_index("c")


    def copy_iw(ci, slot):
        pltpu.make_async_copy(idxh_r.at[wid, :, pl.ds(ci * QC, QC)],
                              idx_v.at[slot], sem_iw.at[slot]).start()
        pltpu.make_async_copy(wgth_r.at[wid, :, pl.ds(ci * QC, QC)],
                              wgt_v.at[slot], sem_iw.at[slot]).start()

    def wait_iw(ci, slot):
        pltpu.make_async_copy(idxh_r.at[wid, :, pl.ds(ci * QC, QC)],
                              idx_v.at[slot], sem_iw.at[slot]).wait()
        pltpu.make_async_copy(wgth_r.at[wid, :, pl.ds(ci * QC, QC)],
                              wgt_v.at[slot], sem_iw.at[slot]).wait()

    def gathers(slot):
        return [pltpu.make_async_copy(vT_r.at[idx_v.at[slot, k]],
                                      rows_v.at[slot, pl.ds(k * QC, QC)],
                                      sem_gat.at[slot])
                for k in range(K)]

    def compute(ci, slot):
        def qbloop(qb, c2):
            wq = [wgt_v[slot, k, pl.ds(qb * 16, 16)] for k in range(K)]
            for qq in range(16):
                qi = qb * 16 + qq
                a0 = wq[0][qq] * rows_v[slot, qi, pl.ds(0, 16)]
                a1 = wq[0][qq] * rows_v[slot, qi, pl.ds(16, 16)]
                b0 = wq[1][qq] * rows_v[slot, QC + qi, pl.ds(0, 16)]
                b1 = wq[1][qq] * rows_v[slot, QC + qi, pl.ds(16, 16)]
                for k in range(2, K, 2):
                    a0 = a0 + wq[k][qq] * rows_v[slot, k * QC + qi, pl.ds(0, 16)]
                    a1 = a1 + wq[k][qq] * rows_v[slot, k * QC + qi, pl.ds(16, 16)]
                    b0 = b0 + wq[k + 1][qq] * rows_v[slot, (k + 1) * QC + qi, pl.ds(0, 16)]
                    b1 = b1 + wq[k + 1][qq] * rows_v[slot, (k + 1) * QC + qi, pl.ds(16, 16)]
                out_v[slot, qi, pl.ds(0, 16)] = a0 + b0
                out_v[slot, qi, pl.ds(16, 16)] = a1 + b1
            return c2

        lax.fori_loop(0, QC // 16, qbloop, 0)
        pltpu.make_async_copy(out_v.at[slot],
                              out_r.at[pl.ds(wid * N + ci * QC, QC)],
                              sem_out.at[slot]).start()

    def wait_out(ci, slot):
        pltpu.make_async_copy(out_v.at[slot],
                              out_r.at[pl.ds(wid * N + ci * QC, QC)],
                              sem_out.at[slot]).wait()

    copy_iw(0, 0)
    copy_iw(1, 1)
    wait_iw(0, 0)
    for cp in gathers(0):
        cp.start()

    def pair(pi, carry):
        for b in range(2):
            ci = pi * 2 + b
            slot = b
            nslot = 1 - b
            @pl.when(ci + 1 < NCHUNK)
            def _():
                wait_iw(ci + 1, nslot)
                for cp in gathers(nslot):
                    cp.start()
            for cp in gathers(slot):
                cp.wait()
            @pl.when(ci >= 2)
            def _():
                wait_out(ci - 2, slot)
            compute(ci, slot)
            @pl.when(ci + 2 < NCHUNK)
            def _():
                copy_iw(ci + 2, slot)
        return carry

    lax.fori_loop(0, NCHUNK // 2, pair, 0)
    wait_out(NCHUNK - 2, 0)
    wait_out(NCHUNK - 1, 1)


def _sc_gather(vT_flat, idx, wgt):
    mesh = plsc.VectorSubcoreMesh(core_axis_name="c", subcore_axis_name="s")
    f = pl.kernel(
        _sc_body,
        out_type=jax.ShapeDtypeStruct((NBH * N, HD), jnp.float32),
        mesh=mesh,
        scratch_types=[
            pltpu.VMEM((2, K, QC), jnp.int32),
            pltpu.VMEM((2, K, QC), jnp.float32),
            pltpu.VMEM((2, QC * K, HD), jnp.float32),
            pltpu.VMEM((2, QC, HD), jnp.float32),
            pltpu.SemaphoreType.DMA((2,)),
            pltpu.SemaphoreType.DMA((2,)),
            pltpu.SemaphoreType.DMA((2,)),
        ],
        compiler_params=pltpu.CompilerParams(use_tc_tiling_on_sc=False),
    )
    return f(vT_flat, idx, wgt)


def _ln(x, g, b):
    mu = jnp.mean(x, 1, keepdims=True)
    var = jnp.mean((x - mu) ** 2, 1, keepdims=True)
    return (x - mu) / jnp.sqrt(var + 1e-5) * g + b


def _stage2_body(o_r, q_r, ow_r, ob_r, n1g_r, n1b_r, fw1_r, fw2_r, n2g_r,
                 n2b_r, out_r):
    x = jnp.concatenate([o_r[0, h] for h in range(NH)], axis=1)
    x = _dot(x, ow_r[...]) + ob_r[...] + q_r[0]
    x = _ln(x, n1g_r[...], n1b_r[...])
    y = _dot(jnp.maximum(_dot(x, fw1_r[...]), 0.0), fw2_r[...]) + x
    out_r[0] = _ln(y, n2g_r[...], n2b_r[...])


def _stage2(out_sc, q, ow, ob, n1g, n1b, fw1, fw2, n2g, n2b):
    nb = N // QB
    wfull = lambda shp: pl.BlockSpec(shp, lambda b, i: (0,) * len(shp))
    return pl.pallas_call(
        _stage2_body,
        grid=(B, nb),
        in_specs=[
            pl.BlockSpec((1, NH, QB, HD), lambda b, i: (b, 0, i, 0)),
            pl.BlockSpec((1, QB, C), lambda b, i: (b, i, 0)),
            wfull((C, C)), wfull((1, C)), wfull((1, C)), wfull((1, C)),
            wfull((C, C)), wfull((C, C)), wfull((1, C)), wfull((1, C)),
        ],
        out_specs=pl.BlockSpec((1, QB, C), lambda b, i: (b, i, 0)),
        out_shape=jax.ShapeDtypeStruct((B, N, C), jnp.float32),
        compiler_params=pltpu.CompilerParams(
            dimension_semantics=("parallel", "parallel")),
    )(out_sc, q, ow, ob, n1g, n1b, fw1, fw2, n2g, n2b)


def kernel(bev_feat, lidar_feat, params):
    q = lidar_feat.transpose(0, 2, 3, 1).reshape(B, N, C)
    value = bev_feat.reshape(B, C, N).transpose(0, 2, 1)
    q_pose = _posembed(params)
    value4 = value.reshape(B, N // 4, 4 * C)
    vTs = [_vproj(value4, params["l%d_val_w" % l], params["l%d_val_b" % l])
           for l in range(NLAYER)]
    for l in range(NLAYER):
        pfx = "l%d_" % l
        off_w = params[pfx + "off_w"]
        off_b = params[pfx + "off_b"]
        owx = off_w[:, 0::2].T
        owy = off_w[:, 1::2].T
        obx = off_b[0::2].reshape(32, 1)
        oby = off_b[1::2].reshape(32, 1)
        idx, wgt = _stage1(
            q, q_pose, owx, owy, obx, oby,
            params[pfx + "attw_w"].T, params[pfx + "attw_b"].reshape(32, 1))
        out_sc = _sc_gather(vTs[l].reshape(NBH * N, HD), idx, wgt)
        q = _stage2(
            out_sc.reshape(B, NH, N, HD), q,
            params[pfx + "out_w"], params[pfx + "out_b"].reshape(1, C),
            params[pfx + "n1_g"].reshape(1, C), params[pfx + "n1_b"].reshape(1, C),
            params[pfx + "ffn_w1"], params[pfx + "ffn_w2"],
            params[pfx + "n2_g"].reshape(1, C), params[pfx + "n2_b"].reshape(1, C))
    return q.reshape(B, H, W, C).transpose(0, 3, 1, 2)


# speedup vs baseline: 1.0079x; 1.0079x over previous
import functools

import jax
import jax.numpy as jnp
from jax import lax
from jax.experimental import pallas as pl
from jax.experimental.pallas import tpu as pltpu
from jax.experimental.pallas import tpu_sc as plsc

B, C, H, W = 4, 256, 64, 64
NH, NP, NLAYER = 8, 4, 2
HD = C // NH
N = H * W
K = NP * 4
QB = 512
QC = 64
NCHUNK = N // QC
NBH = B * NH

_PREC = lax.Precision.DEFAULT


def _dot(a, b):
    return jnp.dot(a, b, preferred_element_type=jnp.float32, precision=_PREC)


def _pos_body(w1x_r, w1y_r, g_r, b_r, w2_r, o_r):
    p = lax.broadcasted_iota(jnp.int32, (N, 1), 0)
    xs = (p % W).astype(jnp.float32) * (1.0 / W)
    ys = (p // W).astype(jnp.float32) * (1.0 / H)
    x = xs * w1x_r[...] + ys * w1y_r[...]
    mu = jnp.mean(x, 0, keepdims=True)
    var = jnp.mean((x - mu) ** 2, 0, keepdims=True)
    x = (x - mu) / jnp.sqrt(var + 1e-5) * g_r[...] + b_r[...]
    x = jnp.maximum(x, 0.0)
    o_r[...] = _dot(x, w2_r[...])


def _posembed(params):
    w1 = params["pos_w1"]
    return pl.pallas_call(
        _pos_body,
        out_shape=jax.ShapeDtypeStruct((N, C), jnp.float32),
    )(w1[0:1], w1[1:2], params["pos_bn_g"].reshape(1, C),
      params["pos_bn_b"].reshape(1, C), params["pos_w2"])


def _vproj_body(val4_r, vw_r, vb_r, vT_r):
    for j in range(4):
        vT_r[:, pl.ds(j * HD, HD)] = (
            _dot(val4_r[0][:, j * C:(j + 1) * C], vw_r[0]) + vb_r[0])


def _vproj(value4, vw, vb):
    vwT = vw.reshape(C, NH, HD).transpose(1, 0, 2)
    vbT = vb.reshape(NH, 1, HD)
    return pl.pallas_call(
        _vproj_body,
        grid=(B, N // QB, NH),
        in_specs=[
            pl.BlockSpec((1, QB // 4, 4 * C), lambda b, i, h: (b, i, 0)),
            pl.BlockSpec((1, C, HD), lambda b, i, h: (h, 0, 0)),
            pl.BlockSpec((1, 1, HD), lambda b, i, h: (h, 0, 0)),
        ],
        out_specs=pl.BlockSpec(
            (QB // 4, 128),
            lambda b, i, h: (b * (NH * N // QB) + h * (N // QB) + i, 0)),
        out_shape=jax.ShapeDtypeStruct((NBH * N // 4, 128), jnp.float32),
        compiler_params=pltpu.CompilerParams(
            dimension_semantics=("parallel", "parallel", "arbitrary")),
    )(value4, vwT, vbT)


def _dotT(wT, x):
    return lax.dot_general(wT, x, (((1,), (1,)), ((), ())),
                           preferred_element_type=jnp.float32,
                           precision=_PREC)


def _stage1_body(q_r, qp_r, owx_r, owy_r, obx_r, oby_r, aww_r, awb_r,
                 idx_r, wgt_r):
    bi = pl.program_id(0)
    ib = pl.program_id(1)
    qp = q_r[0] + qp_r[...]
    offx = _dotT(owx_r[...], qp) + obx_r[...]
    offy = _dotT(owy_r[...], qp) + oby_r[...]
    lg = _dotT(aww_r[...], qp) + awb_r[...]
    m = jnp.max(lg, axis=0, keepdims=True)
    e = jnp.exp(lg - m)
    gi = lax.broadcasted_iota(jnp.int32, (32, 32), 0) // NP
    gj = lax.broadcasted_iota(jnp.int32, (32, 32), 1) // NP
    gmat = (gi == gj).astype(jnp.float32)
    aw = e / _dot(gmat, e)

    p = ib * QB + lax.broadcasted_iota(jnp.int32, (1, QB), 1)
    xsf = (p % W).astype(jnp.float32)
    ysf = (p // W).astype(jnp.float32)
    x = xsf + offx - 0.5
    y = ysf + offy - 0.5
    x0 = jnp.floor(x)
    y0 = jnp.floor(y)
    wx1 = x - x0
    wx0 = 1.0 - wx1
    wy1 = y - y0
    wy0 = 1.0 - wy1
    idx_c = []
    wgt_c = []
    for dy, wy in ((0.0, wy0), (1.0, wy1)):
        for dx, wx in ((0.0, wx0), (1.0, wx1)):
            cx = x0 + dx
            cy = y0 + dy
            valid = ((cx >= 0.0) & (cx <= W - 1) & (cy >= 0.0) & (cy <= H - 1))
            cxi = jnp.clip(cx, 0.0, W - 1).astype(jnp.int32)
            cyi = jnp.clip(cy, 0.0, H - 1).astype(jnp.int32)
            idx_c.append(cyi * W + cxi)
            wgt_c.append(jnp.where(valid, wx * wy * aw, 0.0))
    for h in range(NH):
        s = slice(h * NP, (h + 1) * NP)
        base = (bi * NH + h) * N
        idx_r[h] = jnp.concatenate([t[s] for t in idx_c], axis=0) + base
        wgt_r[h] = jnp.concatenate([t[s] for t in wgt_c], axis=0)


def _stage1(q, q_pose, owx, owy, obx, oby, aww, awb):
    nb = N // QB
    return pl.pallas_call(
        _stage1_body,
        grid=(B, nb),
        in_specs=[
            pl.BlockSpec((1, QB, C), lambda b, i: (b, i, 0)),
            pl.BlockSpec((QB, C), lambda b, i: (i, 0)),
            pl.BlockSpec((32, C), lambda b, i: (0, 0)),
            pl.BlockSpec((32, C), lambda b, i: (0, 0)),
            pl.BlockSpec((32, 1), lambda b, i: (0, 0)),
            pl.BlockSpec((32, 1), lambda b, i: (0, 0)),
            pl.BlockSpec((32, C), lambda b, i: (0, 0)),
            pl.BlockSpec((32, 1), lambda b, i: (0, 0)),
        ],
        out_specs=[
            pl.BlockSpec((NH, K, QB), lambda b, i: (b, 0, i)),
            pl.BlockSpec((NH, K, QB), lambda b, i: (b, 0, i)),
        ],
        out_shape=[
            jax.ShapeDtypeStruct((NBH, K, N), jnp.int32),
            jax.ShapeDtypeStruct((NBH, K, N), jnp.float32),
        ],
        compiler_params=pltpu.CompilerParams(
            dimension_semantics=("parallel", "parallel")),
    )(q, q_pose, owx, owy, obx, oby, aww, awb)


def _sc_body(vT_r, idxh_r, wgth_r, out_r, idx_v, wgt_v, rows_v, out_v,
             sem_iw, sem_gat, sem_out):
    wid = lax.axis_index("s") * 2 + lax.axis_index("c")


    def copy_iw(ci, slot):
        pltpu.make_async_copy(idxh_r.at[wid, :, pl.ds(ci * QC, QC)],
                              idx_v.at[slot], sem_iw.at[slot]).start()
        pltpu.make_async_copy(wgth_r.at[wid, :, pl.ds(ci * QC, QC)],
                              wgt_v.at[slot], sem_iw.at[slot]).start()

    def wait_iw(ci, slot):
        pltpu.make_async_copy(idxh_r.at[wid, :, pl.ds(ci * QC, QC)],
                              idx_v.at[slot], sem_iw.at[slot]).wait()
        pltpu.make_async_copy(wgth_r.at[wid, :, pl.ds(ci * QC, QC)],
                              wgt_v.at[slot], sem_iw.at[slot]).wait()

    def gathers(slot):
        return [pltpu.make_async_copy(vT_r.at[idx_v.at[slot, k]],
                                      rows_v.at[slot, pl.ds(k * QC, QC)],
                                      sem_gat.at[slot])
                for k in range(K)]

    def compute(ci, slot):
        def qbloop(qb, c2):
            wq = [wgt_v[slot, k, pl.ds(qb * 16, 16)] for k in range(K)]
            for qq in range(16):
                qi = qb * 16 + qq
                a0 = wq[0][qq] * rows_v[slot, qi, pl.ds(0, 16)]
                a1 = wq[0][qq] * rows_v[slot, qi, pl.ds(16, 16)]
                b0 = wq[1][qq] * rows_v[slot, QC + qi, pl.ds(0, 16)]
                b1 = wq[1][qq] * rows_v[slot, QC + qi, pl.ds(16, 16)]
                for k in range(2, K, 2):
                    a0 = a0 + wq[k][qq] * rows_v[slot, k * QC + qi, pl.ds(0, 16)]
                    a1 = a1 + wq[k][qq] * rows_v[slot, k * QC + qi, pl.ds(16, 16)]
                    b0 = b0 + wq[k + 1][qq] * rows_v[slot, (k + 1) * QC + qi, pl.ds(0, 16)]
                    b1 = b1 + wq[k + 1][qq] * rows_v[slot, (k + 1) * QC + qi, pl.ds(16, 16)]
                out_v[slot, qi, pl.ds(0, 16)] = a0 + b0
                out_v[slot, qi, pl.ds(16, 16)] = a1 + b1
            return c2

        lax.fori_loop(0, QC // 16, qbloop, 0)
        pltpu.make_async_copy(out_v.at[slot],
                              out_r.at[pl.ds(wid * N + ci * QC, QC)],
                              sem_out.at[slot]).start()

    def wait_out(ci, slot):
        pltpu.make_async_copy(out_v.at[slot],
                              out_r.at[pl.ds(wid * N + ci * QC, QC)],
                              sem_out.at[slot]).wait()

    copy_iw(0, 0)
    copy_iw(1, 1)
    wait_iw(0, 0)
    for cp in gathers(0):
        cp.start()

    def pair(pi, carry):
        for b in range(2):
            ci = pi * 2 + b
            slot = b
            nslot = 1 - b
            for cp in gathers(slot):
                cp.wait()
            @pl.when(ci + 1 < NCHUNK)
            def _():
                wait_iw(ci + 1, nslot)
                for cp in gathers(nslot):
                    cp.start()
            @pl.when(ci >= 2)
            def _():
                wait_out(ci - 2, slot)
            compute(ci, slot)
            @pl.when(ci + 2 < NCHUNK)
            def _():
                copy_iw(ci + 2, slot)
        return carry

    lax.fori_loop(0, NCHUNK // 2, pair, 0)
    wait_out(NCHUNK - 2, 0)
    wait_out(NCHUNK - 1, 1)


def _sc_gather(vT_flat, idx, wgt):
    mesh = plsc.VectorSubcoreMesh(core_axis_name="c", subcore_axis_name="s")
    f = pl.kernel(
        _sc_body,
        out_type=jax.ShapeDtypeStruct((NBH * N, HD), jnp.float32),
        mesh=mesh,
        scratch_types=[
            pltpu.VMEM((2, K, QC), jnp.int32),
            pltpu.VMEM((2, K, QC), jnp.float32),
            pltpu.VMEM((2, QC * K, HD), jnp.float32),
            pltpu.VMEM((2, QC, HD), jnp.float32),
            pltpu.SemaphoreType.DMA((2,)),
            pltpu.SemaphoreType.DMA((2,)),
            pltpu.SemaphoreType.DMA((2,)),
        ],
        compiler_params=pltpu.CompilerParams(use_tc_tiling_on_sc=False),
    )
    return f(vT_flat, idx, wgt)


def _ln(x, g, b):
    mu = jnp.mean(x, 1, keepdims=True)
    var = jnp.mean((x - mu) ** 2, 1, keepdims=True)
    return (x - mu) / jnp.sqrt(var + 1e-5) * g + b


def _stage2_body(o_r, q_r, ow_r, ob_r, n1g_r, n1b_r, fw1_r, fw2_r, n2g_r,
                 n2b_r, out_r):
    x = jnp.concatenate([o_r[0, h] for h in range(NH)], axis=1)
    x = _dot(x, ow_r[...]) + ob_r[...] + q_r[0]
    x = _ln(x, n1g_r[...], n1b_r[...])
    y = _dot(jnp.maximum(_dot(x, fw1_r[...]), 0.0), fw2_r[...]) + x
    out_r[0] = _ln(y, n2g_r[...], n2b_r[...])


def _stage2(out_sc, q, ow, ob, n1g, n1b, fw1, fw2, n2g, n2b):
    nb = N // QB
    wfull = lambda shp: pl.BlockSpec(shp, lambda b, i: (0,) * len(shp))
    return pl.pallas_call(
        _stage2_body,
        grid=(B, nb),
        in_specs=[
            pl.BlockSpec((1, NH, QB, HD), lambda b, i: (b, 0, i, 0)),
            pl.BlockSpec((1, QB, C), lambda b, i: (b, i, 0)),
            wfull((C, C)), wfull((1, C)), wfull((1, C)), wfull((1, C)),
            wfull((C, C)), wfull((C, C)), wfull((1, C)), wfull((1, C)),
        ],
        out_specs=pl.BlockSpec((1, QB, C), lambda b, i: (b, i, 0)),
        out_shape=jax.ShapeDtypeStruct((B, N, C), jnp.float32),
        compiler_params=pltpu.CompilerParams(
            dimension_semantics=("parallel", "parallel")),
    )(out_sc, q, ow, ob, n1g, n1b, fw1, fw2, n2g, n2b)


def kernel(bev_feat, lidar_feat, params):
    q = lidar_feat.transpose(0, 2, 3, 1).reshape(B, N, C)
    value = bev_feat.reshape(B, C, N).transpose(0, 2, 1)
    q_pose = _posembed(params)
    value4 = value.reshape(B, N // 4, 4 * C)
    vTs = [_vproj(value4, params["l%d_val_w" % l], params["l%d_val_b" % l])
           for l in range(NLAYER)]
    for l in range(NLAYER):
        pfx = "l%d_" % l
        off_w = params[pfx + "off_w"]
        off_b = params[pfx + "off_b"]
        owx = off_w[:, 0::2].T
        owy = off_w[:, 1::2].T
        obx = off_b[0::2].reshape(32, 1)
        oby = off_b[1::2].reshape(32, 1)
        idx, wgt = _stage1(
            q, q_pose, owx, owy, obx, oby,
            params[pfx + "attw_w"].T, params[pfx + "attw_b"].reshape(32, 1))
        out_sc = _sc_gather(vTs[l].reshape(NBH * N, HD), idx, wgt)
        q = _stage2(
            out_sc.reshape(B, NH, N, HD), q,
            params[pfx + "out_w"], params[pfx + "out_b"].reshape(1, C),
            params[pfx + "n1_g"].reshape(1, C), params[pfx + "n1_b"].reshape(1, C),
            params[pfx + "ffn_w1"], params[pfx + "ffn_w2"],
            params[pfx + "n2_g"].reshape(1, C), params[pfx + "n2_b"].reshape(1, C))
    return q.reshape(B, H, W, C).transpose(0, 3, 1, 2)
